# Initial kernel scaffold; baseline (speedup 1.0000x reference)
import functools

import jax
import jax.numpy as jnp
from jax import lax
from jax.experimental import pallas as pl
from jax.experimental.pallas import tpu as pltpu
from jax.experimental.pallas import tpu_sc as plsc

N = 10000
E = 320000
D_IN = 128
D_HID = 128
D_OUT = 64

NC = 2
NS = 16
NW = NC * NS

EPW = E // NW
CH = 128
K = -(-EPW // CH)
PADE = K * CH - EPW

PAD_ROWS = 16
NPAD = N + PAD_ROWS
ROWS_PER_TILE = NPAD // NS

DEG_LEN = 10240
DEG_PER_TILE = DEG_LEN // NS

NBUF = 4

_mesh = plsc.VectorSubcoreMesh(core_axis_name="c", subcore_axis_name="s")


@functools.partial(
    pl.kernel,
    out_type=jax.ShapeDtypeStruct((NC, DEG_LEN), jnp.float32),
    mesh=_mesh,
    scratch_types=[
        pltpu.VMEM((K, CH), jnp.int32),
        pltpu.VMEM((CH,), jnp.float32),
        pltpu.VMEM_SHARED((DEG_LEN,), jnp.float32),
    ],
)
def _deg(dstp, ones_hbm, zdeg_hbm, out, idx_v, ones_v, deg_sh):
    c = lax.axis_index("c")
    s = lax.axis_index("s")
    w = c * NS + s
    pltpu.sync_copy(dstp.at[w], idx_v)
    pltpu.sync_copy(ones_hbm, ones_v)
    pltpu.sync_copy(zdeg_hbm.at[pl.ds(s * DEG_PER_TILE, DEG_PER_TILE)],
                    deg_sh.at[pl.ds(s * DEG_PER_TILE, DEG_PER_TILE)])
    plsc.subcore_barrier()
    for k in range(K):
        pltpu.sync_copy(ones_v, deg_sh.at[idx_v.at[k]], add=True)
    plsc.subcore_barrier()
    pltpu.sync_copy(deg_sh.at[pl.ds(s * DEG_PER_TILE, DEG_PER_TILE)],
                    out.at[c, pl.ds(s * DEG_PER_TILE, DEG_PER_TILE)])


@functools.partial(
    pl.kernel,
    out_type=jax.ShapeDtypeStruct((NC, NPAD, D_HID), jnp.float32),
    mesh=_mesh,
    scratch_types=[
        pltpu.VMEM((K, CH), jnp.int32),
        pltpu.VMEM((K, CH), jnp.int32),
        pltpu.VMEM((NBUF, CH, D_HID), jnp.float32),
        pltpu.VMEM_SHARED((NPAD, D_HID), jnp.float32),
        pltpu.SemaphoreType.DMA,
    ],
)
def _agg(srcp, dstp, xwp, zrows_hbm, out, src_v, dst_v, rows_v, agg_sh, gsem):
    c = lax.axis_index("c")
    s = lax.axis_index("s")
    w = c * NS + s
    pltpu.sync_copy(srcp.at[w], src_v)
    pltpu.sync_copy(dstp.at[w], dst_v)
    pltpu.sync_copy(zrows_hbm.at[pl.ds(s * ROWS_PER_TILE, ROWS_PER_TILE)],
                    agg_sh.at[pl.ds(s * ROWS_PER_TILE, ROWS_PER_TILE)])
    plsc.subcore_barrier()
    handles = [None] * K
    for k in range(min(NBUF, K)):
        handles[k] = pltpu.async_copy(xwp.at[src_v.at[k]], rows_v.at[k % NBUF], gsem)
    for k in range(K):
        b = k % NBUF
        handles[k].wait()
        pltpu.sync_copy(rows_v.at[b], agg_sh.at[dst_v.at[k]], add=True)
        nk = k + NBUF
        if nk < K:
            handles[nk] = pltpu.async_copy(xwp.at[src_v.at[nk]], rows_v.at[b], gsem)
    plsc.subcore_barrier()
    pltpu.sync_copy(agg_sh.at[pl.ds(s * ROWS_PER_TILE, ROWS_PER_TILE)],
                    out.at[c, pl.ds(s * ROWS_PER_TILE, ROWS_PER_TILE)])


def _prep_body(x_ref, w1_ref, degp_ref, xwp_ref, dis_ref):
    deg = degp_ref[0] + degp_ref[1] + 1.0
    dis = lax.rsqrt(deg)
    xw = jnp.dot(x_ref[...], w1_ref[...], preferred_element_type=jnp.float32)
    xwp_ref[...] = xw * dis
    dis_ref[...] = dis


_prep = pl.pallas_call(
    _prep_body,
    out_shape=[
        jax.ShapeDtypeStruct((NPAD, D_HID), jnp.float32),
        jax.ShapeDtypeStruct((NPAD, 1), jnp.float32),
    ],
)


def _out_body(aggp_ref, xwp_ref, dis_ref, b1_ref, w2_ref, b2_ref, y_ref):
    a = aggp_ref[0] + aggp_ref[1] + xwp_ref[...]
    t = a * dis_ref[...] + b1_ref[...]
    h = jnp.maximum(t, 0.0)[:N]
    y_ref[...] = jnp.dot(h, w2_ref[...], preferred_element_type=jnp.float32) + b2_ref[...]


_out = pl.pallas_call(
    _out_body,
    out_shape=jax.ShapeDtypeStruct((N, D_OUT), jnp.float32),
)


def kernel(x, edge_index, W1, b1, W2, b2):
    wid = jnp.arange(NW, dtype=jnp.int32)
    pad_src = jnp.broadcast_to(wid[:, None], (NW, PADE))
    pad_dst = jnp.broadcast_to(N + (wid % PAD_ROWS)[:, None], (NW, PADE)).astype(jnp.int32)
    srcp = jnp.concatenate([edge_index[0].reshape(NW, EPW), pad_src], axis=1)
    dstp = jnp.concatenate([edge_index[1].reshape(NW, EPW), pad_dst], axis=1)
    srcp = srcp.reshape(NW, K, CH)
    dstp = dstp.reshape(NW, K, CH)

    ones = jnp.ones((CH,), jnp.float32)
    zdeg = jnp.zeros((DEG_LEN,), jnp.float32)
    zrows = jnp.zeros((NPAD, D_HID), jnp.float32)
    xpad = jnp.concatenate([x, jnp.zeros((PAD_ROWS, D_IN), jnp.float32)], axis=0)

    degp = _deg(dstp, ones, zdeg)
    degp_col = degp[:, :NPAD, None]
    xwp, dis = _prep(xpad, W1, degp_col)
    aggp = _agg(srcp, dstp, xwp, zrows)
    y = _out(aggp, xwp, dis, b1.reshape(1, D_HID), W2, b2.reshape(1, D_OUT))
    return y


# speedup vs baseline: 37.6729x; 37.6729x over previous
import functools

import jax
import jax.numpy as jnp
from jax import lax
from jax.experimental import pallas as pl
from jax.experimental.pallas import tpu as pltpu
from jax.experimental.pallas import tpu_sc as plsc

N = 10000
E = 320000
D_IN = 128
D_HID = 128
D_OUT = 64
DH = D_HID // 2

NC = 2
NS = 16
NW = NC * NS

CH = 125
K1 = E // NW // CH
K2 = E // NS // CH

NPAD = 10240
ROWS_PER_TILE = NPAD // NS

NBUF = 6
GAHEAD = 4
SLAG = 2

_mesh = plsc.VectorSubcoreMesh(core_axis_name="c", subcore_axis_name="s")
_sc_params = pltpu.CompilerParams(use_tc_tiling_on_sc=False)


@functools.partial(
    pl.kernel,
    out_type=jax.ShapeDtypeStruct((NC, NPAD), jnp.float32),
    mesh=_mesh,
    compiler_params=_sc_params,
    scratch_types=[
        pltpu.VMEM((K1, CH), jnp.int32),
        pltpu.VMEM((CH,), jnp.float32),
        pltpu.VMEM_SHARED((NPAD,), jnp.float32),
        pltpu.SemaphoreType.DMA,
    ],
)
def _deg(dstp, ones_hbm, zdeg_hbm, out, idx_v, ones_v, deg_sh, sem):
    c = lax.axis_index("c")
    s = lax.axis_index("s")
    w = c * NS + s
    pltpu.sync_copy(dstp.at[w], idx_v)
    pltpu.sync_copy(ones_hbm, ones_v)
    pltpu.sync_copy(zdeg_hbm.at[pl.ds(s * ROWS_PER_TILE, ROWS_PER_TILE)],
                    deg_sh.at[pl.ds(s * ROWS_PER_TILE, ROWS_PER_TILE)])
    plsc.subcore_barrier()
    for g in range(0, K1, 8):
        hs = [pltpu.async_copy(ones_v, deg_sh.at[idx_v.at[k]], sem, add=True)
              for k in range(g, min(g + 8, K1))]
        for h in hs:
            h.wait()
    plsc.subcore_barrier()
    pltpu.sync_copy(deg_sh.at[pl.ds(s * ROWS_PER_TILE, ROWS_PER_TILE)],
                    out.at[c, pl.ds(s * ROWS_PER_TILE, ROWS_PER_TILE)])


@functools.partial(
    pl.kernel,
    out_type=jax.ShapeDtypeStruct((NC, NPAD, DH), jnp.float32),
    mesh=_mesh,
    compiler_params=_sc_params,
    scratch_types=[
        pltpu.VMEM((K2, CH), jnp.int32),
        pltpu.VMEM((K2, CH), jnp.int32),
        pltpu.VMEM((NBUF, CH, DH), jnp.float32),
        pltpu.VMEM_SHARED((NPAD, DH), jnp.float32),
        pltpu.SemaphoreType.DMA,
        pltpu.SemaphoreType.DMA,
    ],
)
def _agg(srcp, dstp, xwph, zrows_hbm, out, src_v, dst_v, rows_v, agg_sh,
         gsem, ssem):
    c = lax.axis_index("c")
    s = lax.axis_index("s")
    pltpu.sync_copy(srcp.at[s], src_v)
    pltpu.sync_copy(dstp.at[s], dst_v)
    pltpu.sync_copy(zrows_hbm.at[pl.ds(s * ROWS_PER_TILE, ROWS_PER_TILE)],
                    agg_sh.at[pl.ds(s * ROWS_PER_TILE, ROWS_PER_TILE)])
    plsc.subcore_barrier()
    table = xwph.at[c]
    gh = [None] * K2
    sh = [None] * K2
    for k in range(GAHEAD):
        gh[k] = pltpu.async_copy(table.at[src_v.at[k]], rows_v.at[k % NBUF], gsem)
    for k in range(K2):
        b = k % NBUF
        gh[k].wait()
        sh[k] = pltpu.async_copy(rows_v.at[b], agg_sh.at[dst_v.at[k]], ssem,
                                 add=True)
        if k >= SLAG:
            sh[k - SLAG].wait()
        nk = k + GAHEAD
        if nk < K2:
            gh[nk] = pltpu.async_copy(table.at[src_v.at[nk]],
                                      rows_v.at[nk % NBUF], gsem)
    for k in range(max(0, K2 - SLAG), K2):
        sh[k].wait()
    plsc.subcore_barrier()
    pltpu.sync_copy(agg_sh.at[pl.ds(s * ROWS_PER_TILE, ROWS_PER_TILE)],
                    out.at[c, pl.ds(s * ROWS_PER_TILE, ROWS_PER_TILE)])


def _mm_body(x_ref, w1_ref, xw_ref):
    xw_ref[...] = jnp.dot(x_ref[...], w1_ref[...],
                          preferred_element_type=jnp.float32)


_mm = pl.pallas_call(
    _mm_body,
    out_shape=jax.ShapeDtypeStruct((NPAD, D_HID), jnp.float32),
)


def _scale_body(xw_ref, degp_ref, xwph_ref, dis_ref):
    deg = degp_ref[0] + degp_ref[1] + 1.0
    dis = lax.rsqrt(deg)
    xwp = xw_ref[...] * dis
    xwph_ref[0] = xwp[:, :DH]
    xwph_ref[1] = xwp[:, DH:]
    dis_ref[...] = dis


_scale = pl.pallas_call(
    _scale_body,
    out_shape=[
        jax.ShapeDtypeStruct((NC, NPAD, DH), jnp.float32),
        jax.ShapeDtypeStruct((NPAD, 1), jnp.float32),
    ],
)


def _out_body(aggh_ref, xwph_ref, dis_ref, b1_ref, w2_ref, b2_ref, y_ref):
    a = jnp.concatenate(
        [aggh_ref[0] + xwph_ref[0], aggh_ref[1] + xwph_ref[1]], axis=1)
    t = a * dis_ref[...] + b1_ref[...]
    h = jnp.maximum(t, 0.0)[:N]
    y_ref[...] = jnp.dot(h, w2_ref[...],
                         preferred_element_type=jnp.float32) + b2_ref[...]


_out = pl.pallas_call(
    _out_body,
    out_shape=jax.ShapeDtypeStruct((N, D_OUT), jnp.float32),
)


def kernel(x, edge_index, W1, b1, W2, b2):
    dstp1 = edge_index[1].reshape(NW, K1, CH)
    srcp2 = edge_index[0].reshape(NS, K2, CH)
    dstp2 = edge_index[1].reshape(NS, K2, CH)

    ones = jnp.ones((CH,), jnp.float32)
    zdeg = jnp.zeros((NPAD,), jnp.float32)
    zrows = jnp.zeros((NPAD, DH), jnp.float32)
    xpad = jnp.concatenate([x, jnp.zeros((NPAD - N, D_IN), jnp.float32)], axis=0)

    xw = _mm(xpad, W1)
    degp = _deg(dstp1, ones, zdeg)
    degp_col = degp[:, :, None]
    xwph, dis = _scale(xw, degp_col)
    aggh = _agg(srcp2, dstp2, xwph, zrows)
    y = _out(aggh, xwph, dis, b1.reshape(1, D_HID), W2, b2.reshape(1, D_OUT))
    return y


# speedup vs baseline: 39.9574x; 1.0606x over previous
import functools

import jax
import jax.numpy as jnp
from jax import lax
from jax.experimental import pallas as pl
from jax.experimental.pallas import tpu as pltpu
from jax.experimental.pallas import tpu_sc as plsc

N = 10000
E = 320000
D_IN = 128
D_HID = 128
D_OUT = 64
DH = D_HID // 2

NC = 2
NS = 16
NW = NC * NS

CH = 125
K1 = E // NW // CH
K2 = E // NS // CH

NPAD = 10240
ROWS_PER_TILE = NPAD // NS

NBUF = 6
GAHEAD = 4
SLAG = 2

_mesh = plsc.VectorSubcoreMesh(core_axis_name="c", subcore_axis_name="s")
_sc_params = pltpu.CompilerParams(use_tc_tiling_on_sc=False)


@functools.partial(
    pl.kernel,
    out_type=jax.ShapeDtypeStruct((NC, NPAD), jnp.float32),
    mesh=_mesh,
    compiler_params=_sc_params,
    scratch_types=[
        pltpu.VMEM((K1, CH), jnp.int32),
        pltpu.VMEM((CH,), jnp.float32),
        pltpu.VMEM_SHARED((NPAD,), jnp.float32),
        pltpu.SemaphoreType.DMA,
    ],
)
def _deg(dstp, ones_hbm, zdeg_hbm, out, idx_v, ones_v, deg_sh, sem):
    c = lax.axis_index("c")
    s = lax.axis_index("s")
    pltpu.sync_copy(dstp.at[s, pl.ds(c * K1, K1)], idx_v)
    pltpu.sync_copy(ones_hbm, ones_v)
    pltpu.sync_copy(zdeg_hbm,
                    deg_sh.at[pl.ds(s * ROWS_PER_TILE, ROWS_PER_TILE)])
    plsc.subcore_barrier()
    for g in range(0, K1, 8):
        hs = [pltpu.async_copy(ones_v, deg_sh.at[idx_v.at[k]], sem, add=True)
              for k in range(g, min(g + 8, K1))]
        for h in hs:
            h.wait()
    plsc.subcore_barrier()
    pltpu.sync_copy(deg_sh.at[pl.ds(s * ROWS_PER_TILE, ROWS_PER_TILE)],
                    out.at[c, pl.ds(s * ROWS_PER_TILE, ROWS_PER_TILE)])


@functools.partial(
    pl.kernel,
    out_type=jax.ShapeDtypeStruct((NC, NPAD, DH), jnp.float32),
    mesh=_mesh,
    compiler_params=_sc_params,
    scratch_types=[
        pltpu.VMEM((K2, CH), jnp.int32),
        pltpu.VMEM((K2, CH), jnp.int32),
        pltpu.VMEM((NBUF, CH, DH), jnp.float32),
        pltpu.VMEM_SHARED((NPAD, DH), jnp.float32),
        pltpu.SemaphoreType.DMA,
        pltpu.SemaphoreType.DMA,
    ],
)
def _agg(srcp, dstp, xwph, zrows_hbm, out, src_v, dst_v, rows_v, agg_sh,
         gsem, ssem):
    c = lax.axis_index("c")
    s = lax.axis_index("s")
    pltpu.sync_copy(srcp.at[s], src_v)
    pltpu.sync_copy(dstp.at[s], dst_v)
    pltpu.sync_copy(zrows_hbm,
                    agg_sh.at[pl.ds(s * ROWS_PER_TILE, ROWS_PER_TILE)])
    plsc.subcore_barrier()
    table = xwph.at[c]
    gh = [None] * K2
    sh = [None] * K2
    for k in range(GAHEAD):
        gh[k] = pltpu.async_copy(table.at[src_v.at[k]], rows_v.at[k % NBUF], gsem)
    for k in range(K2):
        b = k % NBUF
        gh[k].wait()
        sh[k] = pltpu.async_copy(rows_v.at[b], agg_sh.at[dst_v.at[k]], ssem,
                                 add=True)
        if k >= SLAG:
            sh[k - SLAG].wait()
        nk = k + GAHEAD
        if nk < K2:
            gh[nk] = pltpu.async_copy(table.at[src_v.at[nk]],
                                      rows_v.at[nk % NBUF], gsem)
    for k in range(max(0, K2 - SLAG), K2):
        sh[k].wait()
    plsc.subcore_barrier()
    pltpu.sync_copy(agg_sh.at[pl.ds(s * ROWS_PER_TILE, ROWS_PER_TILE)],
                    out.at[c, pl.ds(s * ROWS_PER_TILE, ROWS_PER_TILE)])


def _mm_body(x_ref, w1_ref, xw_ref):
    xw_ref[...] = jnp.dot(x_ref[...], w1_ref[...],
                          preferred_element_type=jnp.float32)


_mm = pl.pallas_call(
    _mm_body,
    out_shape=jax.ShapeDtypeStruct((N, D_HID), jnp.float32),
)


def _scale_body(xw_ref, degp_ref, xwph_ref, dis_ref):
    deg = degp_ref[0] + degp_ref[1] + 1.0
    dis = lax.rsqrt(deg)
    xwp = xw_ref[...] * dis
    xwph_ref[0] = xwp[:, :DH]
    xwph_ref[1] = xwp[:, DH:]
    dis_ref[...] = dis


_scale = pl.pallas_call(
    _scale_body,
    out_shape=[
        jax.ShapeDtypeStruct((NC, N, DH), jnp.float32),
        jax.ShapeDtypeStruct((N, 1), jnp.float32),
    ],
)


def _out_body(aggh_ref, xwph_ref, dis_ref, b1_ref, w2_ref, b2_ref, y_ref):
    a = jnp.concatenate(
        [aggh_ref[0][:N] + xwph_ref[0], aggh_ref[1][:N] + xwph_ref[1]], axis=1)
    t = a * dis_ref[...] + b1_ref[...]
    h = jnp.maximum(t, 0.0)
    y_ref[...] = jnp.dot(h, w2_ref[...],
                         preferred_element_type=jnp.float32) + b2_ref[...]


_out = pl.pallas_call(
    _out_body,
    out_shape=jax.ShapeDtypeStruct((N, D_OUT), jnp.float32),
)


def kernel(x, edge_index, W1, b1, W2, b2):
    srcp2 = edge_index[0].reshape(NS, K2, CH)
    dstp2 = edge_index[1].reshape(NS, K2, CH)

    ones = jnp.ones((CH,), jnp.float32)
    zdeg = jnp.zeros((ROWS_PER_TILE,), jnp.float32)
    zrows = jnp.zeros((ROWS_PER_TILE, DH), jnp.float32)

    xw = _mm(x, W1)
    degp = _deg(dstp2, ones, zdeg)
    degp_col = degp[:, :N, None]
    xwph, dis = _scale(xw, degp_col)
    aggh = _agg(srcp2, dstp2, xwph, zrows)
    y = _out(aggh, xwph, dis, b1.reshape(1, D_HID), W2, b2.reshape(1, D_OUT))
    return y


# speedup vs baseline: 40.2702x; 1.0078x over previous
import functools

import jax
import jax.numpy as jnp
from jax import lax
from jax.experimental import pallas as pl
from jax.experimental.pallas import tpu as pltpu
from jax.experimental.pallas import tpu_sc as plsc

N = 10000
E = 320000
D_IN = 128
D_HID = 128
D_OUT = 64
DH = D_HID // 2

NC = 2
NS = 16
NW = NC * NS

CH = 125
K1 = E // NW // CH
K2 = E // NS // CH

NPAD = 10240
ROWS_PER_TILE = NPAD // NS

NBUF = 6
GAHEAD = 4
SLAG = 2

_mesh = plsc.VectorSubcoreMesh(core_axis_name="c", subcore_axis_name="s")
_sc_params = pltpu.CompilerParams(use_tc_tiling_on_sc=False)


@functools.partial(
    pl.kernel,
    out_type=jax.ShapeDtypeStruct((NC, NPAD), jnp.float32),
    mesh=_mesh,
    compiler_params=_sc_params,
    scratch_types=[
        pltpu.VMEM((K1, CH), jnp.int32),
        pltpu.VMEM((CH,), jnp.float32),
        pltpu.VMEM_SHARED((NPAD,), jnp.float32),
        pltpu.SemaphoreType.DMA,
    ],
)
def _deg(dstp, ones_hbm, zdeg_hbm, out, idx_v, ones_v, deg_sh, sem):
    c = lax.axis_index("c")
    s = lax.axis_index("s")
    pltpu.sync_copy(dstp.at[s, pl.ds(c * K1, K1)], idx_v)
    pltpu.sync_copy(ones_hbm, ones_v)
    pltpu.sync_copy(zdeg_hbm,
                    deg_sh.at[pl.ds(s * ROWS_PER_TILE, ROWS_PER_TILE)])
    plsc.subcore_barrier()
    hs = [None] * K1
    for k in range(K1):
        hs[k] = pltpu.async_copy(ones_v, deg_sh.at[idx_v.at[k]], sem, add=True)
        if k >= 16:
            hs[k - 16].wait()
    for k in range(K1 - 16, K1):
        hs[k].wait()
    plsc.subcore_barrier()
    pltpu.sync_copy(deg_sh.at[pl.ds(s * ROWS_PER_TILE, ROWS_PER_TILE)],
                    out.at[c, pl.ds(s * ROWS_PER_TILE, ROWS_PER_TILE)])


@functools.partial(
    pl.kernel,
    out_type=jax.ShapeDtypeStruct((NC, NPAD, DH), jnp.float32),
    mesh=_mesh,
    compiler_params=_sc_params,
    scratch_types=[
        pltpu.VMEM((K2, CH), jnp.int32),
        pltpu.VMEM((K2, CH), jnp.int32),
        pltpu.VMEM((NBUF, CH, DH), jnp.float32),
        pltpu.VMEM_SHARED((NPAD, DH), jnp.float32),
        pltpu.SemaphoreType.DMA,
        pltpu.SemaphoreType.DMA,
    ],
)
def _agg(srcp, dstp, xwph, zrows_hbm, out, src_v, dst_v, rows_v, agg_sh,
         gsem, ssem):
    c = lax.axis_index("c")
    s = lax.axis_index("s")
    pltpu.sync_copy(srcp.at[s], src_v)
    pltpu.sync_copy(dstp.at[s], dst_v)
    pltpu.sync_copy(zrows_hbm,
                    agg_sh.at[pl.ds(s * ROWS_PER_TILE, ROWS_PER_TILE)])
    plsc.subcore_barrier()
    table = xwph.at[c]
    gh = [None] * K2
    sh = [None] * K2
    for k in range(GAHEAD):
        gh[k] = pltpu.async_copy(table.at[src_v.at[k]], rows_v.at[k % NBUF], gsem)
    for k in range(K2):
        b = k % NBUF
        gh[k].wait()
        sh[k] = pltpu.async_copy(rows_v.at[b], agg_sh.at[dst_v.at[k]], ssem,
                                 add=True)
        if k >= SLAG:
            sh[k - SLAG].wait()
        nk = k + GAHEAD
        if nk < K2:
            gh[nk] = pltpu.async_copy(table.at[src_v.at[nk]],
                                      rows_v.at[nk % NBUF], gsem)
    for k in range(max(0, K2 - SLAG), K2):
        sh[k].wait()
    plsc.subcore_barrier()
    pltpu.sync_copy(agg_sh.at[pl.ds(s * ROWS_PER_TILE, ROWS_PER_TILE)],
                    out.at[c, pl.ds(s * ROWS_PER_TILE, ROWS_PER_TILE)])


def _mm_body(x_ref, w1_ref, xw_ref):
    xw_ref[...] = jnp.dot(x_ref[...], w1_ref[...],
                          preferred_element_type=jnp.float32)


_mm = pl.pallas_call(
    _mm_body,
    out_shape=jax.ShapeDtypeStruct((N, D_HID), jnp.float32),
)


_RB = 2000


def _scale_body(xw_ref, degp_ref, xwph_ref, dis_ref):
    deg = degp_ref[0] + degp_ref[1] + 1.0
    dis = lax.rsqrt(deg)
    xwp = xw_ref[...] * dis
    xwph_ref[0] = xwp[:, :DH]
    xwph_ref[1] = xwp[:, DH:]
    dis_ref[...] = dis


_scale = pl.pallas_call(
    _scale_body,
    grid=(N // _RB,),
    in_specs=[
        pl.BlockSpec((_RB, D_HID), lambda i: (i, 0)),
        pl.BlockSpec((NC, _RB, 1), lambda i: (0, i, 0)),
    ],
    out_specs=[
        pl.BlockSpec((NC, _RB, DH), lambda i: (0, i, 0)),
        pl.BlockSpec((_RB, 1), lambda i: (i, 0)),
    ],
    out_shape=[
        jax.ShapeDtypeStruct((NC, N, DH), jnp.float32),
        jax.ShapeDtypeStruct((N, 1), jnp.float32),
    ],
)


def _out_body(aggh_ref, xwph_ref, dis_ref, b1_ref, w2_ref, b2_ref, y_ref):
    a = jnp.concatenate(
        [aggh_ref[0] + xwph_ref[0], aggh_ref[1] + xwph_ref[1]], axis=1)
    t = a * dis_ref[...] + b1_ref[...]
    h = jnp.maximum(t, 0.0)
    y_ref[...] = jnp.dot(h, w2_ref[...],
                         preferred_element_type=jnp.float32) + b2_ref[...]


_out = pl.pallas_call(
    _out_body,
    grid=(N // _RB,),
    in_specs=[
        pl.BlockSpec((NC, _RB, DH), lambda i: (0, i, 0)),
        pl.BlockSpec((NC, _RB, DH), lambda i: (0, i, 0)),
        pl.BlockSpec((_RB, 1), lambda i: (i, 0)),
        pl.BlockSpec((1, D_HID), lambda i: (0, 0)),
        pl.BlockSpec((D_HID, D_OUT), lambda i: (0, 0)),
        pl.BlockSpec((1, D_OUT), lambda i: (0, 0)),
    ],
    out_specs=pl.BlockSpec((_RB, D_OUT), lambda i: (i, 0)),
    out_shape=jax.ShapeDtypeStruct((N, D_OUT), jnp.float32),
)


def kernel(x, edge_index, W1, b1, W2, b2):
    srcp2 = edge_index[0].reshape(NS, K2, CH)
    dstp2 = edge_index[1].reshape(NS, K2, CH)

    ones = jnp.ones((CH,), jnp.float32)
    zdeg = jnp.zeros((ROWS_PER_TILE,), jnp.float32)
    zrows = jnp.zeros((ROWS_PER_TILE, DH), jnp.float32)

    xw = _mm(x, W1)
    degp = _deg(dstp2, ones, zdeg)
    degp_col = degp[:, :N, None]
    xwph, dis = _scale(xw, degp_col)
    aggh = _agg(srcp2, dstp2, xwph, zrows)
    y = _out(aggh, xwph, dis, b1.reshape(1, D_HID), W2, b2.reshape(1, D_OUT))
    return y


# speedup vs baseline: 40.4895x; 1.0054x over previous
import functools

import jax
import jax.numpy as jnp
from jax import lax
from jax.experimental import pallas as pl
from jax.experimental.pallas import tpu as pltpu
from jax.experimental.pallas import tpu_sc as plsc

N = 10000
E = 320000
D_IN = 128
D_HID = 128
D_OUT = 64
DH = D_HID // 2

NC = 2
NS = 16
NW = NC * NS

CH = 125
K1 = E // NW // CH
K2 = E // NS // CH

NPAD = 10240
ROWS_PER_TILE = NPAD // NS

NBUF = 6
GAHEAD = 4
SLAG = 2

_mesh = plsc.VectorSubcoreMesh(core_axis_name="c", subcore_axis_name="s")
_sc_params = pltpu.CompilerParams(use_tc_tiling_on_sc=False)


@functools.partial(
    pl.kernel,
    out_type=jax.ShapeDtypeStruct((NC, NPAD), jnp.float32),
    mesh=_mesh,
    compiler_params=_sc_params,
    scratch_types=[
        pltpu.VMEM((K1, CH), jnp.int32),
        pltpu.VMEM((CH,), jnp.float32),
        pltpu.VMEM_SHARED((NPAD,), jnp.float32),
        pltpu.SemaphoreType.DMA,
    ],
)
def _deg(dstp, ones_hbm, zdeg_hbm, out, idx_v, ones_v, deg_sh, sem):
    c = lax.axis_index("c")
    s = lax.axis_index("s")
    pltpu.sync_copy(dstp.at[s, pl.ds(c * K1, K1)], idx_v)
    pltpu.sync_copy(ones_hbm, ones_v)
    pltpu.sync_copy(zdeg_hbm,
                    deg_sh.at[pl.ds(s * ROWS_PER_TILE, ROWS_PER_TILE)])
    plsc.subcore_barrier()
    hs = [None] * K1
    for k in range(K1):
        hs[k] = pltpu.async_copy(ones_v, deg_sh.at[idx_v.at[k]], sem, add=True)
        if k >= 16:
            hs[k - 16].wait()
    for k in range(K1 - 16, K1):
        hs[k].wait()
    plsc.subcore_barrier()
    pltpu.sync_copy(deg_sh.at[pl.ds(s * ROWS_PER_TILE, ROWS_PER_TILE)],
                    out.at[c, pl.ds(s * ROWS_PER_TILE, ROWS_PER_TILE)])


@functools.partial(
    pl.kernel,
    out_type=jax.ShapeDtypeStruct((NC, NPAD, DH), jnp.float32),
    mesh=_mesh,
    compiler_params=_sc_params,
    scratch_types=[
        pltpu.VMEM((K2, CH), jnp.int32),
        pltpu.VMEM((K2, CH), jnp.int32),
        pltpu.VMEM((NBUF, CH, DH), jnp.float32),
        pltpu.VMEM_SHARED((NPAD, DH), jnp.float32),
        pltpu.SemaphoreType.DMA,
        pltpu.SemaphoreType.DMA,
    ],
)
def _agg(srcp, dstp, xwph, zrows_hbm, out, src_v, dst_v, rows_v, agg_sh,
         gsem, ssem):
    c = lax.axis_index("c")
    s = lax.axis_index("s")
    pltpu.sync_copy(srcp.at[s], src_v)
    pltpu.sync_copy(dstp.at[s], dst_v)
    pltpu.sync_copy(zrows_hbm,
                    agg_sh.at[pl.ds(s * ROWS_PER_TILE, ROWS_PER_TILE)])
    plsc.subcore_barrier()
    table = xwph.at[c]
    gh = [None] * K2
    sh = [None] * K2
    for k in range(GAHEAD):
        gh[k] = pltpu.async_copy(table.at[src_v.at[k]], rows_v.at[k % NBUF], gsem)
    for k in range(K2):
        b = k % NBUF
        gh[k].wait()
        sh[k] = pltpu.async_copy(rows_v.at[b], agg_sh.at[dst_v.at[k]], ssem,
                                 add=True)
        if k >= SLAG:
            sh[k - SLAG].wait()
        nk = k + GAHEAD
        if nk < K2:
            gh[nk] = pltpu.async_copy(table.at[src_v.at[nk]],
                                      rows_v.at[nk % NBUF], gsem)
    for k in range(max(0, K2 - SLAG), K2):
        sh[k].wait()
    plsc.subcore_barrier()
    pltpu.sync_copy(agg_sh.at[pl.ds(s * ROWS_PER_TILE, ROWS_PER_TILE)],
                    out.at[c, pl.ds(s * ROWS_PER_TILE, ROWS_PER_TILE)])


_RB = 2000


def _prep_body(x_ref, w1_ref, degp_ref, xwph_ref, dis_ref):
    deg = degp_ref[0] + degp_ref[1] + 1.0
    dis = lax.rsqrt(deg)
    xw = jnp.dot(x_ref[...], w1_ref[...], preferred_element_type=jnp.float32)
    xwp = xw * dis
    xwph_ref[0] = xwp[:, :DH]
    xwph_ref[1] = xwp[:, DH:]
    dis_ref[...] = dis


_prep = pl.pallas_call(
    _prep_body,
    grid=(N // _RB,),
    in_specs=[
        pl.BlockSpec((_RB, D_IN), lambda i: (i, 0)),
        pl.BlockSpec((D_IN, D_HID), lambda i: (0, 0)),
        pl.BlockSpec((NC, _RB, 1), lambda i: (0, i, 0)),
    ],
    out_specs=[
        pl.BlockSpec((NC, _RB, DH), lambda i: (0, i, 0)),
        pl.BlockSpec((_RB, 1), lambda i: (i, 0)),
    ],
    out_shape=[
        jax.ShapeDtypeStruct((NC, N, DH), jnp.float32),
        jax.ShapeDtypeStruct((N, 1), jnp.float32),
    ],
)


def _out_body(aggh_ref, xwph_ref, dis_ref, b1_ref, w2_ref, b2_ref, y_ref):
    a = jnp.concatenate(
        [aggh_ref[0] + xwph_ref[0], aggh_ref[1] + xwph_ref[1]], axis=1)
    t = a * dis_ref[...] + b1_ref[...]
    h = jnp.maximum(t, 0.0)
    y_ref[...] = jnp.dot(h, w2_ref[...],
                         preferred_element_type=jnp.float32) + b2_ref[...]


_out = pl.pallas_call(
    _out_body,
    grid=(N // _RB,),
    in_specs=[
        pl.BlockSpec((NC, _RB, DH), lambda i: (0, i, 0)),
        pl.BlockSpec((NC, _RB, DH), lambda i: (0, i, 0)),
        pl.BlockSpec((_RB, 1), lambda i: (i, 0)),
        pl.BlockSpec((1, D_HID), lambda i: (0, 0)),
        pl.BlockSpec((D_HID, D_OUT), lambda i: (0, 0)),
        pl.BlockSpec((1, D_OUT), lambda i: (0, 0)),
    ],
    out_specs=pl.BlockSpec((_RB, D_OUT), lambda i: (i, 0)),
    out_shape=jax.ShapeDtypeStruct((N, D_OUT), jnp.float32),
)


def kernel(x, edge_index, W1, b1, W2, b2):
    srcp2 = edge_index[0].reshape(NS, K2, CH)
    dstp2 = edge_index[1].reshape(NS, K2, CH)

    ones = jnp.ones((CH,), jnp.float32)
    zdeg = jnp.zeros((ROWS_PER_TILE,), jnp.float32)
    zrows = jnp.zeros((ROWS_PER_TILE, DH), jnp.float32)

    degp = _deg(dstp2, ones, zdeg)
    degp_col = degp[:, :N, None]
    xwph, dis = _prep(x, W1, degp_col)
    aggh = _agg(srcp2, dstp2, xwph, zrows)
    y = _out(aggh, xwph, dis, b1.reshape(1, D_HID), W2, b2.reshape(1, D_OUT))
    return y


# speedup vs baseline: 41.2011x; 1.0176x over previous
import functools

import jax
import jax.numpy as jnp
from jax import lax
from jax.experimental import pallas as pl
from jax.experimental.pallas import tpu as pltpu
from jax.experimental.pallas import tpu_sc as plsc

N = 10000
E = 320000
D_IN = 128
D_HID = 128
D_OUT = 64
DH = D_HID // 2

NC = 2
NS = 16
NW = NC * NS

CH = 125
K1 = E // NW // CH
K2 = E // NS // CH

NPAD = 10240
ROWS_PER_TILE = NPAD // NS

NBUF = 6
GAHEAD = 4
SLAG = 2

_mesh = plsc.VectorSubcoreMesh(core_axis_name="c", subcore_axis_name="s")
_sc_params = pltpu.CompilerParams(use_tc_tiling_on_sc=False)


@functools.partial(
    pl.kernel,
    out_type=jax.ShapeDtypeStruct((NC, NPAD), jnp.float32),
    mesh=_mesh,
    compiler_params=_sc_params,
    scratch_types=[
        pltpu.VMEM((K1, CH), jnp.int32),
        pltpu.VMEM((CH,), jnp.float32),
        pltpu.VMEM_SHARED((NPAD,), jnp.float32),
        pltpu.SemaphoreType.DMA,
    ],
)
def _deg(dstp, ones_hbm, zdeg_hbm, out, idx_v, ones_v, deg_sh, sem):
    c = lax.axis_index("c")
    s = lax.axis_index("s")
    pltpu.sync_copy(dstp.at[s, pl.ds(c * K1, K1)], idx_v)
    pltpu.sync_copy(ones_hbm, ones_v)
    pltpu.sync_copy(zdeg_hbm,
                    deg_sh.at[pl.ds(s * ROWS_PER_TILE, ROWS_PER_TILE)])
    plsc.subcore_barrier()
    hs = [None] * K1
    for k in range(K1):
        hs[k] = pltpu.async_copy(ones_v, deg_sh.at[idx_v.at[k]], sem, add=True)
        if k >= 16:
            hs[k - 16].wait()
    for k in range(K1 - 16, K1):
        hs[k].wait()
    plsc.subcore_barrier()
    pltpu.sync_copy(deg_sh.at[pl.ds(s * ROWS_PER_TILE, ROWS_PER_TILE)],
                    out.at[c, pl.ds(s * ROWS_PER_TILE, ROWS_PER_TILE)])


@functools.partial(
    pl.kernel,
    out_type=jax.ShapeDtypeStruct((NC, NPAD, DH), jnp.float32),
    mesh=_mesh,
    compiler_params=_sc_params,
    scratch_types=[
        pltpu.VMEM((K2, CH), jnp.int32),
        pltpu.VMEM((K2, CH), jnp.int32),
        pltpu.VMEM((NBUF, CH, DH), jnp.float32),
        pltpu.VMEM_SHARED((NPAD, DH), jnp.float32),
        pltpu.SemaphoreType.DMA,
        pltpu.SemaphoreType.DMA,
    ],
)
def _agg(srcp, dstp, xwph, zrows_hbm, out, src_v, dst_v, rows_v, agg_sh,
         gsem, ssem):
    c = lax.axis_index("c")
    s = lax.axis_index("s")
    pltpu.sync_copy(srcp.at[s], src_v)
    pltpu.sync_copy(dstp.at[s], dst_v)
    table = xwph.at[c]
    if True:
        lo = s * ROWS_PER_TILE

        @pl.when(s < NS - 1)
        def _():
            pltpu.sync_copy(table.at[pl.ds(lo, ROWS_PER_TILE)],
                            agg_sh.at[pl.ds(lo, ROWS_PER_TILE)])

        @pl.when(s == NS - 1)
        def _():
            pltpu.sync_copy(table.at[pl.ds(N - 400, 400)],
                            agg_sh.at[pl.ds(N - 400, 400)])
            pltpu.sync_copy(zrows_hbm, agg_sh.at[pl.ds(N, NPAD - N)])
    plsc.subcore_barrier()
    gh = [None] * K2
    sh = [None] * K2
    for k in range(GAHEAD):
        gh[k] = pltpu.async_copy(table.at[src_v.at[k]], rows_v.at[k % NBUF], gsem)
    for k in range(K2):
        b = k % NBUF
        gh[k].wait()
        sh[k] = pltpu.async_copy(rows_v.at[b], agg_sh.at[dst_v.at[k]], ssem,
                                 add=True)
        if k >= SLAG:
            sh[k - SLAG].wait()
        nk = k + GAHEAD
        if nk < K2:
            gh[nk] = pltpu.async_copy(table.at[src_v.at[nk]],
                                      rows_v.at[nk % NBUF], gsem)
    for k in range(max(0, K2 - SLAG), K2):
        sh[k].wait()
    plsc.subcore_barrier()
    pltpu.sync_copy(agg_sh.at[pl.ds(s * ROWS_PER_TILE, ROWS_PER_TILE)],
                    out.at[c, pl.ds(s * ROWS_PER_TILE, ROWS_PER_TILE)])


_RB = 2000


def _prep_body(x_ref, w1_ref, degp_ref, xwph_ref, dis_ref):
    deg = degp_ref[0] + degp_ref[1] + 1.0
    dis = lax.rsqrt(deg)
    xw = jnp.dot(x_ref[...], w1_ref[...], preferred_element_type=jnp.float32)
    xwp = xw * dis
    xwph_ref[0] = xwp[:, :DH]
    xwph_ref[1] = xwp[:, DH:]
    dis_ref[...] = dis


_prep = pl.pallas_call(
    _prep_body,
    grid=(N // _RB,),
    in_specs=[
        pl.BlockSpec((_RB, D_IN), lambda i: (i, 0)),
        pl.BlockSpec((D_IN, D_HID), lambda i: (0, 0)),
        pl.BlockSpec((NC, _RB, 1), lambda i: (0, i, 0)),
    ],
    out_specs=[
        pl.BlockSpec((NC, _RB, DH), lambda i: (0, i, 0)),
        pl.BlockSpec((_RB, 1), lambda i: (i, 0)),
    ],
    out_shape=[
        jax.ShapeDtypeStruct((NC, N, DH), jnp.float32),
        jax.ShapeDtypeStruct((N, 1), jnp.float32),
    ],
)


def _out_body(aggh_ref, dis_ref, b1_ref, w2_ref, b2_ref, y_ref):
    t0 = aggh_ref[0] * dis_ref[...] + b1_ref[:, :DH]
    t1 = aggh_ref[1] * dis_ref[...] + b1_ref[:, DH:]
    h0 = jnp.maximum(t0, 0.0)
    h1 = jnp.maximum(t1, 0.0)
    y_ref[...] = (jnp.dot(h0, w2_ref[:DH], preferred_element_type=jnp.float32)
                  + jnp.dot(h1, w2_ref[DH:], preferred_element_type=jnp.float32)
                  + b2_ref[...])


_out = pl.pallas_call(
    _out_body,
    grid=(N // _RB,),
    in_specs=[
        pl.BlockSpec((NC, _RB, DH), lambda i: (0, i, 0)),
        pl.BlockSpec((_RB, 1), lambda i: (i, 0)),
        pl.BlockSpec((1, D_HID), lambda i: (0, 0)),
        pl.BlockSpec((D_HID, D_OUT), lambda i: (0, 0)),
        pl.BlockSpec((1, D_OUT), lambda i: (0, 0)),
    ],
    out_specs=pl.BlockSpec((_RB, D_OUT), lambda i: (i, 0)),
    out_shape=jax.ShapeDtypeStruct((N, D_OUT), jnp.float32),
)


def kernel(x, edge_index, W1, b1, W2, b2):
    srcp2 = edge_index[0].reshape(NS, K2, CH)
    dstp2 = edge_index[1].reshape(NS, K2, CH)

    ones = jnp.ones((CH,), jnp.float32)
    zdeg = jnp.zeros((ROWS_PER_TILE,), jnp.float32)
    zrows = jnp.zeros((NPAD - N, DH), jnp.float32)

    degp = _deg(dstp2, ones, zdeg)
    degp_col = degp[:, :N, None]
    xwph, dis = _prep(x, W1, degp_col)
    aggh = _agg(srcp2, dstp2, xwph, zrows)
    y = _out(aggh, dis, b1.reshape(1, D_HID), W2, b2.reshape(1, D_OUT))
    return y


# speedup vs baseline: 43.2514x; 1.0498x over previous
import functools

import jax
import jax.numpy as jnp
from jax import lax
from jax.experimental import pallas as pl
from jax.experimental.pallas import tpu as pltpu
from jax.experimental.pallas import tpu_sc as plsc

N = 10000
E = 320000
D_IN = 128
D_HID = 128
D_OUT = 64
DH = D_HID // 2

NC = 2
NS = 16
NW = NC * NS

CH = 125
K1 = E // NW // CH
K2 = E // NS // CH

NPAD = 10240
ROWS_PER_TILE = NPAD // NS

NBUF = 6
GAHEAD = 4
SLAG = 2

_mesh = plsc.VectorSubcoreMesh(core_axis_name="c", subcore_axis_name="s")
_sc_params = pltpu.CompilerParams(use_tc_tiling_on_sc=False)


@functools.partial(
    pl.kernel,
    out_type=jax.ShapeDtypeStruct((NC, NPAD), jnp.float32),
    mesh=_mesh,
    compiler_params=_sc_params,
    scratch_types=[
        pltpu.VMEM((K1, CH), jnp.int32),
        pltpu.VMEM((CH,), jnp.float32),
        pltpu.VMEM_SHARED((NPAD,), jnp.float32),
        pltpu.SemaphoreType.DMA,
    ],
)
def _deg(dstp, ones_hbm, zdeg_hbm, out, idx_v, ones_v, deg_sh, sem):
    c = lax.axis_index("c")
    s = lax.axis_index("s")
    pltpu.sync_copy(dstp.at[s, pl.ds(c * K1, K1)], idx_v)
    pltpu.sync_copy(ones_hbm, ones_v)
    pltpu.sync_copy(zdeg_hbm,
                    deg_sh.at[pl.ds(s * ROWS_PER_TILE, ROWS_PER_TILE)])
    plsc.subcore_barrier()
    hs = [None] * K1
    for k in range(K1):
        hs[k] = pltpu.async_copy(ones_v, deg_sh.at[idx_v.at[k]], sem, add=True)
        if k >= 16:
            hs[k - 16].wait()
    for k in range(K1 - 16, K1):
        hs[k].wait()
    plsc.subcore_barrier()
    pltpu.sync_copy(deg_sh.at[pl.ds(s * ROWS_PER_TILE, ROWS_PER_TILE)],
                    out.at[c, pl.ds(s * ROWS_PER_TILE, ROWS_PER_TILE)])


@functools.partial(
    pl.kernel,
    out_type=jax.ShapeDtypeStruct((NC, NPAD, DH), jnp.float32),
    mesh=_mesh,
    compiler_params=_sc_params,
    scratch_types=[
        pltpu.VMEM((K2, CH), jnp.int32),
        pltpu.VMEM((K2, CH), jnp.int32),
        pltpu.VMEM((NBUF, CH, DH), jnp.float32),
        pltpu.VMEM_SHARED((NPAD, DH), jnp.float32),
        pltpu.SemaphoreType.DMA,
        pltpu.SemaphoreType.DMA,
    ],
)
def _agg(srcp, dstp, xwph, zrows_hbm, out, src_v, dst_v, rows_v, agg_sh,
         gsem, ssem):
    c = lax.axis_index("c")
    s = lax.axis_index("s")
    pltpu.sync_copy(srcp.at[s], src_v)
    pltpu.sync_copy(dstp.at[s], dst_v)
    table = xwph.at[c]
    if True:
        lo = s * ROWS_PER_TILE

        @pl.when(s < NS - 1)
        def _():
            pltpu.sync_copy(table.at[pl.ds(lo, ROWS_PER_TILE)],
                            agg_sh.at[pl.ds(lo, ROWS_PER_TILE)])

        @pl.when(s == NS - 1)
        def _():
            pltpu.sync_copy(table.at[pl.ds(N - 400, 400)],
                            agg_sh.at[pl.ds(N - 400, 400)])
            pltpu.sync_copy(zrows_hbm, agg_sh.at[pl.ds(N, NPAD - N)])
    plsc.subcore_barrier()
    gh = [None] * K2
    sh = [None] * K2
    for k in range(GAHEAD):
        gh[k] = pltpu.async_copy(table.at[src_v.at[k]], rows_v.at[k % NBUF], gsem)
    for k in range(K2):
        b = k % NBUF
        gh[k].wait()
        sh[k] = pltpu.async_copy(rows_v.at[b], agg_sh.at[dst_v.at[k]], ssem,
                                 add=True)
        if k >= SLAG:
            sh[k - SLAG].wait()
        nk = k + GAHEAD
        if nk < K2:
            gh[nk] = pltpu.async_copy(table.at[src_v.at[nk]],
                                      rows_v.at[nk % NBUF], gsem)
    for k in range(max(0, K2 - SLAG), K2):
        sh[k].wait()
    plsc.subcore_barrier()
    pltpu.sync_copy(agg_sh.at[pl.ds(s * ROWS_PER_TILE, ROWS_PER_TILE)],
                    out.at[c, pl.ds(s * ROWS_PER_TILE, ROWS_PER_TILE)])


_RB = 2000


def _prep_body(x_ref, w1_ref, degp_ref, xwph_ref, dis_ref):
    deg = degp_ref[0] + degp_ref[1] + 1.0
    dis = lax.rsqrt(deg)
    xw = jnp.dot(x_ref[...], w1_ref[...], preferred_element_type=jnp.float32)
    xwp = xw * dis
    xwph_ref[0] = xwp[:, :DH]
    xwph_ref[1] = xwp[:, DH:]
    dis_ref[...] = dis


_prep = pl.pallas_call(
    _prep_body,
    grid=(N // _RB,),
    in_specs=[
        pl.BlockSpec((_RB, D_IN), lambda i: (i, 0)),
        pl.BlockSpec((D_IN, D_HID), lambda i: (0, 0)),
        pl.BlockSpec((NC, _RB, 1), lambda i: (0, i, 0)),
    ],
    out_specs=[
        pl.BlockSpec((NC, _RB, DH), lambda i: (0, i, 0)),
        pl.BlockSpec((_RB, 1), lambda i: (i, 0)),
    ],
    out_shape=[
        jax.ShapeDtypeStruct((NC, N, DH), jnp.float32),
        jax.ShapeDtypeStruct((N, 1), jnp.float32),
    ],
)


_CRB = _RB // 2


def _out_body(aggc_ref, dis2_ref, b1_ref, w2_ref, b2c_ref, y_ref):
    d_even = jnp.broadcast_to(dis2_ref[:, 0:1], (_CRB, DH))
    d_odd = jnp.broadcast_to(dis2_ref[:, 1:2], (_CRB, DH))
    disc = jnp.concatenate([d_even, d_odd], axis=1)
    b1lo = jnp.concatenate([b1_ref[:, :DH], b1_ref[:, :DH]], axis=1)
    b1hi = jnp.concatenate([b1_ref[:, DH:], b1_ref[:, DH:]], axis=1)
    h0 = jnp.maximum(aggc_ref[0] * disc + b1lo, 0.0)
    h1 = jnp.maximum(aggc_ref[1] * disc + b1hi, 0.0)
    w2lo, w2hi = w2_ref[:DH], w2_ref[DH:]
    y_even = (jnp.dot(h0[:, :DH], w2lo, preferred_element_type=jnp.float32)
              + jnp.dot(h1[:, :DH], w2hi, preferred_element_type=jnp.float32))
    y_odd = (jnp.dot(h0[:, DH:], w2lo, preferred_element_type=jnp.float32)
             + jnp.dot(h1[:, DH:], w2hi, preferred_element_type=jnp.float32))
    y_ref[...] = jnp.concatenate([y_even, y_odd], axis=1) + b2c_ref[...]


_out = pl.pallas_call(
    _out_body,
    grid=(N // _RB,),
    in_specs=[
        pl.BlockSpec((NC, _CRB, D_HID), lambda i: (0, i, 0)),
        pl.BlockSpec((_CRB, 2), lambda i: (i, 0)),
        pl.BlockSpec((1, D_HID), lambda i: (0, 0)),
        pl.BlockSpec((D_HID, D_OUT), lambda i: (0, 0)),
        pl.BlockSpec((1, 2 * D_OUT), lambda i: (0, 0)),
    ],
    out_specs=pl.BlockSpec((_CRB, 2 * D_OUT), lambda i: (i, 0)),
    out_shape=jax.ShapeDtypeStruct((N // 2, 2 * D_OUT), jnp.float32),
)


def kernel(x, edge_index, W1, b1, W2, b2):
    srcp2 = edge_index[0].reshape(NS, K2, CH)
    dstp2 = edge_index[1].reshape(NS, K2, CH)

    ones = jnp.ones((CH,), jnp.float32)
    zdeg = jnp.zeros((ROWS_PER_TILE,), jnp.float32)
    zrows = jnp.zeros((NPAD - N, DH), jnp.float32)

    degp = _deg(dstp2, ones, zdeg)
    degp_col = degp[:, :N, None]
    xwph, dis = _prep(x, W1, degp_col)
    aggh = _agg(srcp2, dstp2, xwph, zrows)
    aggc = aggh.reshape(NC, NPAD // 2, 2 * DH)
    dis2 = dis.reshape(N // 2, 2)
    b2c = jnp.concatenate([b2, b2]).reshape(1, 2 * D_OUT)
    yc = _out(aggc, dis2, b1.reshape(1, D_HID), W2, b2c)
    return yc.reshape(N, D_OUT)
